# Initial kernel scaffold; baseline (speedup 1.0000x reference)
import jax
import jax.numpy as jnp
from jax.experimental import pallas as pl


def kernel(ref_feats, src_feats):
    raise NotImplementedError("write your pallas kernel here")


# speedup vs baseline: 46.6916x; 46.6916x over previous
import jax
import jax.numpy as jnp
from jax import lax
from jax.experimental import pallas as pl
from jax.experimental.pallas import tpu as pltpu
from jax.experimental.pallas import tpu_sc as plsc

M, N, K = 4096, 16384, 256
BM, BN = 512, 2048
MT, NT = M // BM, N // BN
TOPK = 512
GRP = 16
NGRP = (M * N) // GRP
NCHUNK = (M * N) // 128
GPR = N // GRP

NWORK = 16
RPW = M // NWORK
RWIN = 8
NWIN = RPW // RWIN
DCAP = 20480
GB = 128
CCAP = 24576


def _tc_body(a_ref, b_ref, s_ref, gmax_ref, rmax_ref, rarg_ref):
    j = pl.program_id(1)
    xy = lax.dot_general(a_ref[...], b_ref[...], (((1,), (1,)), ((), ())),
                         preferred_element_type=jnp.float32)
    s = jnp.exp(-(2.0 - 2.0 * xy))
    s_ref[...] = s.reshape(BM, BN // 128, 128)
    gmax_ref[...] = jnp.max(s.reshape(BM, BN // GRP, GRP), axis=2)
    tm = jnp.max(s, axis=1)
    colid = lax.broadcasted_iota(jnp.int32, (BM, BN), 1)
    targ = jnp.min(jnp.where(s == tm[:, None], colid, BN), axis=1)
    tmax = tm.reshape(1, 1, BM)
    targ = (targ.astype(jnp.int32) + j * BN).reshape(1, 1, BM)

    @pl.when(j == 0)
    def _():
        rmax_ref[...] = tmax
        rarg_ref[...] = targ

    @pl.when(j != 0)
    def _():
        old = rmax_ref[...]
        upd = tmax > old
        rmax_ref[...] = jnp.where(upd, tmax, old)
        rarg_ref[...] = jnp.where(upd, targ, rarg_ref[...])


_tc_call = pl.pallas_call(
    _tc_body,
    grid=(MT, NT),
    in_specs=[
        pl.BlockSpec((BM, K), lambda i, j: (i, 0)),
        pl.BlockSpec((BN, K), lambda i, j: (j, 0)),
    ],
    out_specs=[
        pl.BlockSpec((BM, BN // 128, 128), lambda i, j: (i, j, 0)),
        pl.BlockSpec((BM, BN // GRP), lambda i, j: (i, j)),
        pl.BlockSpec((1, 1, BM), lambda i, j: (i, 0, 0)),
        pl.BlockSpec((1, 1, BM), lambda i, j: (i, 0, 0)),
    ],
    out_shape=[
        jax.ShapeDtypeStruct((M, 128, 128), jnp.float32),
        jax.ShapeDtypeStruct((M, GPR), jnp.float32),
        jax.ShapeDtypeStruct((MT, 1, BM), jnp.float32),
        jax.ShapeDtypeStruct((MT, 1, BM), jnp.int32),
    ],
)


def _iota16():
    return lax.broadcasted_iota(jnp.int32, (16,), 0)


def _vsum(x):
    return lax.reduce_sum_p.bind(x, axes=(0,))


def _full(v):
    return jnp.full((16,), v, jnp.int32)


_STAGE = 4


def _sc_body(s_hbm, gmax_hbm, thr_hbm,
             refi_hbm, srci_hbm, score_hbm,
             gwin_v, gath_v, gids_v, cids_v, ckey_v, cidx_v,
             thr_v, stage_v, row_v, wreg_v,
             gk_v, gi_v, ek_v, outr_v, outs_v, outv_v,
             sh_cnt, sh_gk, sh_gi, sh_ei,
             sem0):
    wid = lax.axis_index("s")
    iota = _iota16()

    pltpu.sync_copy(thr_hbm, thr_v)
    thrv = thr_v[...]

    row0w = wid * RPW

    def win_loop(w, nd):
        r0 = row0w + w * RWIN
        pltpu.sync_copy(gmax_hbm.at[pl.ds(r0, RWIN)], gwin_v)

        def row_loop(r, nd):
            def vreg_loop(k, nd):
                g = gwin_v[r, pl.ds(k * 16, 16)]
                m = g >= thrv
                pc = _vsum(jnp.where(m, 1, 0).astype(jnp.int32))

                @pl.when(pc > 0)
                def _():
                    gid = (r0 + r) * GPR + k * 16 + iota
                    plsc.store_compressed(gids_v.at[pl.ds(nd, 16)], gid,
                                          mask=m)
                    plsc.store_compressed(cids_v.at[pl.ds(nd, 16)],
                                          lax.shift_right_logical(gid, 3),
                                          mask=m)

                return jnp.minimum(nd + pc, DCAP)

            return lax.fori_loop(0, GPR // 16, vreg_loop, nd)

        return lax.fori_loop(0, RWIN, row_loop, nd)

    nd = lax.fori_loop(0, NWIN, win_loop, jnp.int32(0))

    def pad_loop(k, _):
        cids_v[pl.ds(nd + k * 16, 16)] = jnp.zeros((16,), jnp.int32)
        return 0

    lax.fori_loop(0, GB // 16, pad_loop, 0)

    def _diag(val):
        @pl.when(wid == 0)
        def _():
            stage_v[...] = _full(val)
            pltpu.sync_copy(stage_v, refi_hbm.at[pl.ds(0, 16)])

    if _STAGE == 1:
        _diag(nd)
        return

    nbatch = (nd + GB - 1) // GB

    def batch_loop(b, nc):
        pltpu.async_copy(
            s_hbm.at[cids_v.at[pl.ds(b * GB, GB)]], gath_v, sem0).wait()
        nb = jnp.minimum(nd - b * GB, GB)

        def grp_loop(jl, nc):
            j = b * GB + jl
            gid = plsc.load_gather(gids_v, [_full(j)])
            col = (gid & 7) * 16 + iota
            v = plsc.load_gather(gath_v, [_full(jl), col])
            m = v >= thrv
            pc = _vsum(jnp.where(m, 1, 0).astype(jnp.int32))

            @pl.when(pc > 0)
            def _():
                fidx = gid * GRP + iota
                plsc.store_compressed(ckey_v.at[pl.ds(nc, 16)],
                                      plsc.bitcast(v, jnp.int32), mask=m)
                plsc.store_compressed(cidx_v.at[pl.ds(nc, 16)], fidx, mask=m)

            return jnp.minimum(nc + pc, CCAP)

        return lax.fori_loop(0, nb, grp_loop, nc)

    nc = lax.fori_loop(0, nbatch, batch_loop, jnp.int32(0))
    nv = (nc + 15) // 16

    if _STAGE == 2:
        _diag(nc)
        return

    def count_ge(t):
        tv = _full(t)

        def cnt_loop(k, acc):
            kv = ckey_v[pl.ds(k * 16, 16)]
            valid = (k * 16 + iota) < nc
            m = (kv >= tv) & valid
            return acc + jnp.where(m, 1, 0).astype(jnp.int32)

        return _vsum(lax.fori_loop(0, nv, cnt_loop,
                                   jnp.zeros((16,), jnp.int32)))

    def publish_and_total(val):
        stage_v[...] = jnp.where(iota == 0, _full(val), 0)
        plsc.subcore_barrier()
        pltpu.sync_copy(stage_v, sh_cnt.at[wid])
        plsc.subcore_barrier()
        pltpu.sync_copy(sh_cnt, wreg_v)

        def srow(w, acc):
            return acc + _vsum(wreg_v[w])

        return lax.fori_loop(0, NWORK, srow, jnp.int32(0))

    def bs_round(r, t):
        t2 = t | (jnp.int32(1) << (30 - r))
        total = publish_and_total(count_ge(t2))
        return jnp.where(total >= TOPK, t2, t)

    tstar = lax.fori_loop(0, 31, bs_round, jnp.int32(0))
    tsv = _full(tstar)

    if _STAGE == 3:
        _diag(tstar)
        return

    def cmp_loop(k, carry):
        ng, ne = carry
        kv = ckey_v[pl.ds(k * 16, 16)]
        iv = cidx_v[pl.ds(k * 16, 16)]
        valid = (k * 16 + iota) < nc
        mg = (kv > tsv) & valid
        me = (kv == tsv) & valid
        pg = _vsum(jnp.where(mg, 1, 0).astype(jnp.int32))
        pe = _vsum(jnp.where(me, 1, 0).astype(jnp.int32))

        @pl.when(pg > 0)
        def _():
            plsc.store_compressed(gk_v.at[pl.ds(ng, 16)], kv, mask=mg)
            plsc.store_compressed(gi_v.at[pl.ds(ng, 16)], iv, mask=mg)

        @pl.when(pe > 0)
        def _():
            plsc.store_compressed(ek_v.at[pl.ds(ne, 16)], iv, mask=me)

        return (jnp.minimum(ng + pg, TOPK), jnp.minimum(ne + pe, TOPK))

    ng, ne = lax.fori_loop(0, nv, cmp_loop, (jnp.int32(0), jnp.int32(0)))

    stage_v[...] = jnp.where(iota == 0, _full(ng),
                             jnp.where(iota == 1, _full(ne), 0))
    plsc.subcore_barrier()
    pltpu.sync_copy(stage_v, sh_cnt.at[wid])
    pltpu.sync_copy(gk_v.at[pl.ds(0, TOPK)], sh_gk.at[wid])
    pltpu.sync_copy(gi_v.at[pl.ds(0, TOPK)], sh_gi.at[wid])
    pltpu.sync_copy(ek_v.at[pl.ds(0, TOPK)], sh_ei.at[wid])
    plsc.subcore_barrier()

    @pl.when(wid == 0)
    def _():
        pltpu.sync_copy(sh_cnt, wreg_v)

        def meta(w):
            row = wreg_v[w]
            g = _vsum(jnp.where(iota == 0, row, 0))
            e = _vsum(jnp.where(iota == 1, row, 0))
            return jnp.clip(g, 0, TOPK), jnp.clip(e, 0, TOPK)

        def gath_gt(w, cnt):
            pltpu.sync_copy(sh_gk.at[w], gk_v.at[pl.ds(0, TOPK)])
            pltpu.sync_copy(sh_gi.at[w], gi_v.at[pl.ds(0, TOPK)])
            gw, _ = meta(w)

            def cp(k, cnt):
                kv = gk_v[pl.ds(k * 16, 16)]
                iv = gi_v[pl.ds(k * 16, 16)]
                m = (k * 16 + iota) < gw
                pc = _vsum(jnp.where(m, 1, 0).astype(jnp.int32))

                @pl.when(pc > 0)
                def _():
                    plsc.store_compressed(ckey_v.at[pl.ds(cnt, 16)], kv,
                                          mask=m)
                    plsc.store_compressed(cidx_v.at[pl.ds(cnt, 16)], iv,
                                          mask=m)

                return cnt + pc

            return lax.fori_loop(0, (gw + 15) // 16, cp, cnt)

        c1 = jnp.minimum(lax.fori_loop(0, NWORK, gath_gt, jnp.int32(0)),
                         TOPK)

        def place_gt(i, _):
            isp = _full(i)
            ki = plsc.load_gather(ckey_v, [isp])
            vi = plsc.load_gather(cidx_v, [isp])

            def rk(k, acc):
                kv = ckey_v[pl.ds(k * 16, 16)]
                pos = k * 16 + iota
                gtm = (kv > ki) & (pos < c1)
                eqm = (kv == ki) & (pos < i)
                return (acc + jnp.where(gtm, 1, 0).astype(jnp.int32)
                        + jnp.where(eqm, 1, 0).astype(jnp.int32))

            rank = jnp.minimum(
                _vsum(lax.fori_loop(0, (c1 + 15) // 16, rk,
                                    jnp.zeros((16,), jnp.int32))), TOPK - 1)
            one = iota == 0
            plsc.store_scatter(outv_v, [_full(rank)], ki, mask=one)
            plsc.store_scatter(outr_v, [_full(rank)], vi, mask=one)
            return 0

        lax.fori_loop(0, c1, place_gt, 0)

        def gath_eq(w, pos):
            pltpu.sync_copy(sh_ei.at[w], ek_v.at[pl.ds(0, TOPK)])
            _, ew = meta(w)
            take = jnp.clip(TOPK - pos, 0, ew)

            def cp(k, pos):
                iv = ek_v[pl.ds(k * 16, 16)]
                m = (k * 16 + iota) < take
                pc = _vsum(jnp.where(m, 1, 0).astype(jnp.int32))

                @pl.when(pc > 0)
                def _():
                    plsc.store_scatter(outv_v, [pos + iota], tsv, mask=m)
                    plsc.store_scatter(outr_v, [pos + iota], iv, mask=m)

                return pos + pc

            return lax.fori_loop(0, (take + 15) // 16, cp, pos)

        lax.fori_loop(0, NWORK, gath_eq, c1)

        def emit(k, _):
            fi = outr_v[pl.ds(k * 16, 16)]
            kv = outv_v[pl.ds(k * 16, 16)]
            row_v[...] = lax.shift_right_logical(fi, 14)
            pltpu.sync_copy(row_v, refi_hbm.at[pl.ds(k * 16, 16)])
            row_v[...] = fi & jnp.int32(N - 1)
            pltpu.sync_copy(row_v, srci_hbm.at[pl.ds(k * 16, 16)])
            outs_v[...] = plsc.bitcast(kv, jnp.float32)
            pltpu.sync_copy(outs_v, score_hbm.at[pl.ds(k * 16, 16)])
            return 0

        lax.fori_loop(0, TOPK // 16, emit, 0)


def _sc_call(s_chunks, gmax, thr):
    mesh = plsc.VectorSubcoreMesh(core_axis_name="c", subcore_axis_name="s",
                                  num_cores=2, num_subcores=NWORK)
    f = pl.kernel(
        _sc_body,
        out_type=(
            jax.ShapeDtypeStruct((TOPK,), jnp.int32),
            jax.ShapeDtypeStruct((TOPK,), jnp.int32),
            jax.ShapeDtypeStruct((TOPK,), jnp.float32),
        ),
        mesh=mesh,
        compiler_params=pltpu.CompilerParams(needs_layout_passes=False),
        scratch_types=[
            pltpu.VMEM((RWIN, GPR), jnp.float32),
            pltpu.VMEM((GB, 128), jnp.float32),
            pltpu.VMEM((DCAP + GB + 16,), jnp.int32),
            pltpu.VMEM((DCAP + GB + 16,), jnp.int32),
            pltpu.VMEM((CCAP + 16,), jnp.int32),
            pltpu.VMEM((CCAP + 16,), jnp.int32),
            pltpu.VMEM((16,), jnp.float32),
            pltpu.VMEM((16,), jnp.int32),
            pltpu.VMEM((16,), jnp.int32),
            pltpu.VMEM((NWORK, 16), jnp.int32),
            pltpu.VMEM((TOPK + 16,), jnp.int32),
            pltpu.VMEM((TOPK + 16,), jnp.int32),
            pltpu.VMEM((TOPK + 16,), jnp.int32),
            pltpu.VMEM((TOPK,), jnp.int32),
            pltpu.VMEM((16,), jnp.float32),
            pltpu.VMEM((TOPK,), jnp.int32),
            pltpu.VMEM_SHARED((NWORK, 16), jnp.int32),
            pltpu.VMEM_SHARED((NWORK, TOPK), jnp.int32),
            pltpu.VMEM_SHARED((NWORK, TOPK), jnp.int32),
            pltpu.VMEM_SHARED((NWORK, TOPK), jnp.int32),
            pltpu.SemaphoreType.DMA,
        ],
    )
    return f(s_chunks, gmax, thr)


@jax.jit
def kernel(ref_feats, src_feats):
    s3, gmax, rmax, rarg = _tc_call(ref_feats, src_feats)
    v0 = jnp.min(rmax)
    thr = jnp.full((16,), v0, jnp.float32)
    refi, srci, scores = _sc_call(s3.reshape(NCHUNK, 128), gmax, thr)
    return refi, srci, scores, rarg.reshape(M)
